# Initial kernel scaffold; baseline (speedup 1.0000x reference)
import jax
import jax.numpy as jnp
from jax.experimental import pallas as pl


def kernel(xyz, feature, W1, b1, Wl1, bl1, Wl2, bl2, Wsc, bsc):
    raise NotImplementedError("write your pallas kernel here")


# speedup vs baseline: 14.4020x; 14.4020x over previous
import functools

import jax
import jax.numpy as jnp
from jax import lax
from jax.experimental import pallas as pl
from jax.experimental.pallas import tpu as pltpu
from jax.experimental.pallas import tpu_sc as plsc

K = 16
TN = 256
TNE = 512


def _topk_body(xyz_t_ref, xyz_c_ref, idx_ref, *, n):
    b = pl.program_id(0)
    rows = xyz_t_ref[0]
    allm = xyz_c_ref[0]
    mm = jnp.dot(rows, allm, preferred_element_type=jnp.float32)
    sqr = jnp.sum(rows * rows, axis=1, keepdims=True)
    sqa = jnp.sum(allm * allm, axis=0, keepdims=True)
    negd = -((-2.0 * mm + sqr) + sqa)
    iota = lax.broadcasted_iota(jnp.int32, negd.shape, 1)
    cols = []
    for _ in range(K):
        m = jnp.max(negd, axis=1, keepdims=True)
        am = jnp.min(jnp.where(negd == m, iota, jnp.int32(n)),
                     axis=1, keepdims=True)
        cols.append(am)
        negd = jnp.where(iota == am, -jnp.inf, negd)
    idx_ref[0] = jnp.concatenate(cols, axis=1) + b * n


def _topk_idx(xyz_t8, xyz_c8):
    b, n, _ = xyz_t8.shape
    grid = (b, n // TN)
    return pl.pallas_call(
        functools.partial(_topk_body, n=n),
        grid=grid,
        in_specs=[
            pl.BlockSpec((1, TN, 8), lambda bi, i: (bi, i, 0)),
            pl.BlockSpec((1, 8, n), lambda bi, i: (bi, 0, 0)),
        ],
        out_specs=pl.BlockSpec((1, TN, K), lambda bi, i: (bi, i, 0)),
        out_shape=jax.ShapeDtypeStruct((b, n, K), jnp.int32),
    )(xyz_t8, xyz_c8)


def _sc_gather(idx_flat, table):
    rows = idx_flat.shape[0]
    nc, ns = 2, 16
    nw = nc * ns
    rpw = rows // nw
    mesh = plsc.VectorSubcoreMesh(core_axis_name="c", subcore_axis_name="s")

    @functools.partial(
        pl.kernel, mesh=mesh,
        out_type=jax.ShapeDtypeStruct((rows, 16), jnp.float32),
        compiler_params=pltpu.CompilerParams(use_tc_tiling_on_sc=False),
        scratch_types=[
            pltpu.VMEM((rpw,), jnp.int32),
            pltpu.VMEM((rpw, 16), jnp.float32),
            pltpu.SemaphoreType.DMA,
        ],
    )
    def k(idx_hbm, table_hbm, out_hbm, idx_v, rows_v, sem):
        wid = lax.axis_index("s") * nc + lax.axis_index("c")
        base = wid * rpw
        pltpu.sync_copy(idx_hbm.at[pl.ds(base, rpw)], idx_v)
        pltpu.async_copy(table_hbm.at[idx_v], rows_v, sem).wait()
        pltpu.sync_copy(rows_v, out_hbm.at[pl.ds(base, rpw)])

    return k(idx_flat, table)


def _encode_body(knn_ref, cen_ref, feat_ref, w1t_ref, wl1t_ref, wl2t_ref,
                 b1_ref, bl1_ref, bl2_ref, wsct_ref, bsc_ref, out_ref, *, h):
    f32 = jnp.float32
    wl1t = wl1t_ref[...]
    wl2t = wl2t_ref[...]
    w_eff = jnp.dot(jnp.dot(w1t_ref[...], wl1t, preferred_element_type=f32),
                    wl2t, preferred_element_type=f32)
    b_eff = jnp.dot(jnp.dot(b1_ref[...], wl1t, preferred_element_type=f32)
                    + bl1_ref[...], wl2t, preferred_element_type=f32) \
        + bl2_ref[...]
    wc = w_eff[0:3]
    wk = w_eff[3:6]
    wd = w_eff[6:9]
    ww = w_eff[9:10]
    zpad = jnp.zeros((13, h), f32)
    m_kd = jnp.concatenate([wk + wd, zpad], axis=0)
    m_cd = jnp.concatenate([wc - wd, zpad], axis=0)

    knn = knn_ref[0]
    cen = cen_ref[0]
    cen_rep = jnp.broadcast_to(cen[:, None, :],
                               (TNE, K, 16)).reshape(TNE * K, 16)
    diff = knn - cen_rep
    d2 = jnp.sum(diff * diff, axis=1, keepdims=True)
    hk = jnp.dot(knn, m_kd, preferred_element_type=f32) + d2 * ww
    maxh = jnp.max(hk.reshape(TNE, K, h), axis=1)
    enc = maxh + jnp.dot(cen, m_cd, preferred_element_type=f32) + b_eff
    sc = jnp.dot(feat_ref[0], wsct_ref[...],
                 preferred_element_type=f32) + bsc_ref[...]
    out_ref[0] = jnp.concatenate([enc, enc], axis=1) + sc


def _encode(knn, xyz_t16, feat_t, w1t16, wl1t, wl2t, b1r, bl1r, bl2r,
            wsct, bscr):
    b, n, _ = xyz_t16.shape
    h = wl1t.shape[0]
    d_out = wsct.shape[1]
    grid = (b, n // TNE)
    zero2 = lambda bi, i: (0, 0)
    return pl.pallas_call(
        functools.partial(_encode_body, h=h),
        grid=grid,
        in_specs=[
            pl.BlockSpec((1, TNE * K, 16), lambda bi, i: (bi, i, 0)),
            pl.BlockSpec((1, TNE, 16), lambda bi, i: (bi, i, 0)),
            pl.BlockSpec((1, TNE, feat_t.shape[2]), lambda bi, i: (bi, i, 0)),
            pl.BlockSpec((16, h), zero2),
            pl.BlockSpec((h, h), zero2),
            pl.BlockSpec((h, h), zero2),
            pl.BlockSpec((1, h), zero2),
            pl.BlockSpec((1, h), zero2),
            pl.BlockSpec((1, h), zero2),
            pl.BlockSpec((feat_t.shape[2], d_out), zero2),
            pl.BlockSpec((1, d_out), zero2),
        ],
        out_specs=pl.BlockSpec((1, TNE, d_out), lambda bi, i: (bi, i, 0)),
        out_shape=jax.ShapeDtypeStruct((b, n, d_out), jnp.float32),
    )(knn, xyz_t16, feat_t, w1t16, wl1t, wl2t, b1r, bl1r, bl2r, wsct, bscr)


def kernel(xyz, feature, W1, b1, Wl1, bl1, Wl2, bl2, Wsc, bsc):
    b, _, n = xyz.shape
    xyz_t = jnp.transpose(xyz, (0, 2, 1))
    xyz_t16 = jnp.pad(xyz_t, ((0, 0), (0, 0), (0, 13)))
    xyz_c8 = jnp.pad(xyz, ((0, 0), (0, 5), (0, 0)))
    xyz_t8 = xyz_t16[..., :8]

    idx = _topk_idx(xyz_t8, xyz_c8)
    table = xyz_t16.reshape(b * n, 16)
    knn_flat = _sc_gather(idx.reshape(-1), table)
    knn = knn_flat.reshape(b, n * K, 16)

    feat_t = jnp.transpose(feature, (0, 2, 1))
    h = W1.shape[0]
    w1t16 = jnp.pad(W1.T, ((0, 6), (0, 0)))
    out_nc = _encode(knn, xyz_t16, feat_t, w1t16, Wl1.T, Wl2.T,
                     b1[None, :], bl1[None, :], bl2[None, :],
                     Wsc.T, bsc[None, :])
    return jnp.transpose(out_nc, (0, 2, 1))


# speedup vs baseline: 16.6291x; 1.1546x over previous
import functools

import jax
import jax.numpy as jnp
from jax import lax
from jax.experimental import pallas as pl
from jax.experimental.pallas import tpu as pltpu
from jax.experimental.pallas import tpu_sc as plsc

K = 16
TN = 256
TNE = 512


def _topk_body(xyz_t_ref, xyz_c_ref, idx_ref, *, n):
    b = pl.program_id(0)
    rows = xyz_t_ref[0]
    allm = xyz_c_ref[0]
    mm = jnp.dot(rows, allm, preferred_element_type=jnp.float32)
    sqr = jnp.sum(rows * rows, axis=1, keepdims=True)
    sqa = jnp.sum(allm * allm, axis=0, keepdims=True)
    negd = -((-2.0 * mm + sqr) + sqa)
    iota = lax.broadcasted_iota(jnp.int32, negd.shape, 1)
    cols = []
    for _ in range(K):
        am = jnp.argmax(negd, axis=1).astype(jnp.int32)[:, None]
        cols.append(am)
        negd = jnp.where(iota == am, -jnp.inf, negd)
    idx_ref[0] = jnp.concatenate(cols, axis=1) + b * n


def _topk_idx(xyz_t8, xyz_c8):
    b, n, _ = xyz_t8.shape
    grid = (b, n // TN)
    return pl.pallas_call(
        functools.partial(_topk_body, n=n),
        grid=grid,
        in_specs=[
            pl.BlockSpec((1, TN, 8), lambda bi, i: (bi, i, 0)),
            pl.BlockSpec((1, 8, n), lambda bi, i: (bi, 0, 0)),
        ],
        out_specs=pl.BlockSpec((1, TN, K), lambda bi, i: (bi, i, 0)),
        out_shape=jax.ShapeDtypeStruct((b, n, K), jnp.int32),
    )(xyz_t8, xyz_c8)


def _sc_gather(idx_flat, table):
    rows = idx_flat.shape[0]
    nc, ns = 2, 16
    nw = nc * ns
    rpw = rows // nw
    mesh = plsc.VectorSubcoreMesh(core_axis_name="c", subcore_axis_name="s")

    @functools.partial(
        pl.kernel, mesh=mesh,
        out_type=jax.ShapeDtypeStruct((rows, 16), jnp.float32),
        compiler_params=pltpu.CompilerParams(use_tc_tiling_on_sc=False),
        scratch_types=[
            pltpu.VMEM((rpw,), jnp.int32),
            pltpu.VMEM((rpw, 16), jnp.float32),
            pltpu.SemaphoreType.DMA,
        ],
    )
    def k(idx_hbm, table_hbm, out_hbm, idx_v, rows_v, sem):
        wid = lax.axis_index("s") * nc + lax.axis_index("c")
        base = wid * rpw
        pltpu.sync_copy(idx_hbm.at[pl.ds(base, rpw)], idx_v)
        pltpu.async_copy(table_hbm.at[idx_v], rows_v, sem).wait()
        pltpu.sync_copy(rows_v, out_hbm.at[pl.ds(base, rpw)])

    return k(idx_flat, table)


def _encode_body(knn_ref, cen_ref, feat_ref, w1t_ref, wl1t_ref, wl2t_ref,
                 b1_ref, bl1_ref, bl2_ref, wsc_ref, bsc_ref, out_ref, *, h):
    f32 = jnp.float32
    wl1t = wl1t_ref[...]
    wl2t = wl2t_ref[...]
    w_eff = jnp.dot(jnp.dot(w1t_ref[...], wl1t, preferred_element_type=f32),
                    wl2t, preferred_element_type=f32)
    b_eff = jnp.dot(jnp.dot(b1_ref[...], wl1t, preferred_element_type=f32)
                    + bl1_ref[...], wl2t, preferred_element_type=f32) \
        + bl2_ref[...]
    wc = w_eff[0:3]
    wk = w_eff[3:6]
    wd = w_eff[6:9]
    ww = w_eff[9:10]
    zpad = jnp.zeros((13, h), f32)
    m_kd = jnp.concatenate([wk + wd, zpad], axis=0)
    m_cd = jnp.concatenate([wc - wd, zpad], axis=0)

    knn = knn_ref[0]
    cen = cen_ref[0]
    cen_rep = jnp.broadcast_to(cen[:, None, :],
                               (TNE, K, 16)).reshape(TNE * K, 16)
    diff = knn - cen_rep
    d2 = jnp.sum(diff * diff, axis=1, keepdims=True)
    hk = jnp.dot(knn, m_kd, preferred_element_type=f32) + d2 * ww
    maxh = jnp.max(hk.reshape(TNE, K, h), axis=1)
    enc = maxh + jnp.dot(cen, m_cd, preferred_element_type=f32) + b_eff
    enc_t = jnp.transpose(enc)
    sc_t = jnp.dot(wsc_ref[...], feat_ref[0],
                   preferred_element_type=f32) + bsc_ref[...]
    out_ref[0] = jnp.concatenate([enc_t, enc_t], axis=0) + sc_t


def _encode(knn, xyz_t16, feature, w1t16, wl1t, wl2t, b1r, bl1r, bl2r,
            wsc, bscc):
    b, n, _ = xyz_t16.shape
    h = wl1t.shape[0]
    d_in = feature.shape[1]
    d_out = wsc.shape[0]
    grid = (b, n // TNE)
    zero2 = lambda bi, i: (0, 0)
    return pl.pallas_call(
        functools.partial(_encode_body, h=h),
        grid=grid,
        in_specs=[
            pl.BlockSpec((1, TNE * K, 16), lambda bi, i: (bi, i, 0)),
            pl.BlockSpec((1, TNE, 16), lambda bi, i: (bi, i, 0)),
            pl.BlockSpec((1, d_in, TNE), lambda bi, i: (bi, 0, i)),
            pl.BlockSpec((16, h), zero2),
            pl.BlockSpec((h, h), zero2),
            pl.BlockSpec((h, h), zero2),
            pl.BlockSpec((1, h), zero2),
            pl.BlockSpec((1, h), zero2),
            pl.BlockSpec((1, h), zero2),
            pl.BlockSpec((d_out, d_in), zero2),
            pl.BlockSpec((d_out, 1), zero2),
        ],
        out_specs=pl.BlockSpec((1, d_out, TNE), lambda bi, i: (bi, 0, i)),
        out_shape=jax.ShapeDtypeStruct((b, d_out, n), jnp.float32),
    )(knn, xyz_t16, feature, w1t16, wl1t, wl2t, b1r, bl1r, bl2r, wsc, bscc)


def kernel(xyz, feature, W1, b1, Wl1, bl1, Wl2, bl2, Wsc, bsc):
    b, _, n = xyz.shape
    xyz_t = jnp.transpose(xyz, (0, 2, 1))
    xyz_t16 = jnp.pad(xyz_t, ((0, 0), (0, 0), (0, 13)))
    xyz_c8 = jnp.pad(xyz, ((0, 0), (0, 5), (0, 0)))
    xyz_t8 = xyz_t16[..., :8]

    idx = _topk_idx(xyz_t8, xyz_c8)
    table = xyz_t16.reshape(b * n, 16)
    knn_flat = _sc_gather(idx.reshape(-1), table)
    knn = knn_flat.reshape(b, n * K, 16)

    w1t16 = jnp.pad(W1.T, ((0, 6), (0, 0)))
    return _encode(knn, xyz_t16, feature, w1t16, Wl1.T, Wl2.T,
                   b1[None, :], bl1[None, :], bl2[None, :],
                   Wsc, bsc[:, None])


# speedup vs baseline: 16.6374x; 1.0005x over previous
import functools

import jax
import jax.numpy as jnp
from jax import lax
from jax.experimental import pallas as pl
from jax.experimental.pallas import tpu as pltpu
from jax.experimental.pallas import tpu_sc as plsc

K = 16
TN = 256
TNE = 512


def _topk_body(xyz_t_ref, xyz_c_ref, idx_ref, *, n):
    b = pl.program_id(0)
    rows = xyz_t_ref[0]
    allm = xyz_c_ref[0]
    mm = jnp.dot(rows, allm, preferred_element_type=jnp.float32)
    sqr = jnp.sum(rows * rows, axis=1, keepdims=True)
    sqa = jnp.sum(allm * allm, axis=0, keepdims=True)
    negd = -((-2.0 * mm + sqr) + sqa)
    iota = lax.broadcasted_iota(jnp.int32, negd.shape, 1)
    cols = []
    for _ in range(K):
        am = jnp.argmax(negd, axis=1).astype(jnp.int32)[:, None]
        cols.append(am)
        negd = jnp.where(iota == am, -jnp.inf, negd)
    idx_ref[0] = jnp.concatenate(cols, axis=1) + b * n


def _topk_idx(xyz_t8, xyz_c8):
    b, n, _ = xyz_t8.shape
    grid = (b, n // TN)
    return pl.pallas_call(
        functools.partial(_topk_body, n=n),
        grid=grid,
        in_specs=[
            pl.BlockSpec((1, TN, 8), lambda bi, i: (bi, i, 0)),
            pl.BlockSpec((1, 8, n), lambda bi, i: (bi, 0, 0)),
        ],
        out_specs=pl.BlockSpec((1, TN, K), lambda bi, i: (bi, i, 0)),
        out_shape=jax.ShapeDtypeStruct((b, n, K), jnp.int32),
    )(xyz_t8, xyz_c8)


def _sc_gather(idx_flat, table):
    rows = idx_flat.shape[0]
    nc, ns = 2, 16
    nw = nc * ns
    rpw = rows // nw
    mesh = plsc.VectorSubcoreMesh(core_axis_name="c", subcore_axis_name="s")

    @functools.partial(
        pl.kernel, mesh=mesh,
        out_type=jax.ShapeDtypeStruct((rows, 16), jnp.float32),
        compiler_params=pltpu.CompilerParams(use_tc_tiling_on_sc=False),
        scratch_types=[
            pltpu.VMEM((rpw,), jnp.int32),
            pltpu.VMEM((rpw, 16), jnp.float32),
            pltpu.SemaphoreType.DMA,
        ],
    )
    def k(idx_hbm, table_hbm, out_hbm, idx_v, rows_v, sem):
        wid = lax.axis_index("s") * nc + lax.axis_index("c")
        base = wid * rpw
        pltpu.sync_copy(idx_hbm.at[pl.ds(base, rpw)], idx_v)
        pltpu.async_copy(table_hbm.at[idx_v], rows_v, sem).wait()
        pltpu.sync_copy(rows_v, out_hbm.at[pl.ds(base, rpw)])

    return k(idx_flat, table)


def _encode_body(knn_ref, cen_ref, feat_ref, w1t_ref, wl1t_ref, wl2t_ref,
                 b1_ref, bl1_ref, bl2_ref, wsc_ref, bsc_ref, out_ref, *, h):
    f32 = jnp.float32
    wl1t = wl1t_ref[...]
    wl2t = wl2t_ref[...]
    w_eff = jnp.dot(jnp.dot(w1t_ref[...], wl1t, preferred_element_type=f32),
                    wl2t, preferred_element_type=f32)
    b_eff = jnp.dot(jnp.dot(b1_ref[...], wl1t, preferred_element_type=f32)
                    + bl1_ref[...], wl2t, preferred_element_type=f32) \
        + bl2_ref[...]
    wc = w_eff[0:3]
    wk = w_eff[3:6]
    wd = w_eff[6:9]
    ww = w_eff[9:10]
    m_kd = jnp.concatenate([wk + wd, jnp.zeros((13, h), f32)], axis=0)
    m_cd = jnp.concatenate([wc - wd, jnp.zeros((13, h), f32)], axis=0)

    knn = knn_ref[0]
    cen = cen_ref[0]
    cen_rep = jnp.broadcast_to(cen[:, None, :],
                               (TNE, K, 16)).reshape(TNE * K, 16)
    diff = knn - cen_rep
    d2 = jnp.sum(diff * diff, axis=1, keepdims=True)
    hk = jnp.dot(knn, m_kd, preferred_element_type=f32) + d2 * ww
    maxh = jnp.max(hk.reshape(TNE, K, h), axis=1)
    enc = maxh + jnp.dot(cen, m_cd, preferred_element_type=f32) + b_eff
    enc_t = jnp.transpose(enc)
    sc_t = jnp.dot(wsc_ref[...], feat_ref[0],
                   preferred_element_type=f32) + bsc_ref[...]
    out_ref[0] = jnp.concatenate([enc_t, enc_t], axis=0) + sc_t


def _encode(knn, xyz_t16, feature, w1t16, wl1t, wl2t, b1r, bl1r, bl2r,
            wsc, bscc):
    b, n, _ = xyz_t16.shape
    h = wl1t.shape[0]
    d_in = feature.shape[1]
    d_out = wsc.shape[0]
    grid = (b, n // TNE)
    zero2 = lambda bi, i: (0, 0)
    return pl.pallas_call(
        functools.partial(_encode_body, h=h),
        grid=grid,
        in_specs=[
            pl.BlockSpec((1, TNE * K, 16), lambda bi, i: (bi, i, 0)),
            pl.BlockSpec((1, TNE, 16), lambda bi, i: (bi, i, 0)),
            pl.BlockSpec((1, d_in, TNE), lambda bi, i: (bi, 0, i)),
            pl.BlockSpec((16, h), zero2),
            pl.BlockSpec((h, h), zero2),
            pl.BlockSpec((h, h), zero2),
            pl.BlockSpec((1, h), zero2),
            pl.BlockSpec((1, h), zero2),
            pl.BlockSpec((1, h), zero2),
            pl.BlockSpec((d_out, d_in), zero2),
            pl.BlockSpec((d_out, 1), zero2),
        ],
        out_specs=pl.BlockSpec((1, d_out, TNE), lambda bi, i: (bi, 0, i)),
        out_shape=jax.ShapeDtypeStruct((b, d_out, n), jnp.float32),
    )(knn, xyz_t16, feature, w1t16, wl1t, wl2t, b1r, bl1r, bl2r, wsc, bscc)


def kernel(xyz, feature, W1, b1, Wl1, bl1, Wl2, bl2, Wsc, bsc):
    b, _, n = xyz.shape
    xyz_t = jnp.transpose(xyz, (0, 2, 1))
    xyz_t16 = jnp.pad(xyz_t, ((0, 0), (0, 0), (0, 13)))
    xyz_c8 = jnp.pad(xyz, ((0, 0), (0, 5), (0, 0)))
    xyz_t8 = xyz_t16[..., :8]

    idx = _topk_idx(xyz_t8, xyz_c8)
    table = xyz_t16.reshape(b * n, 16)
    knn_flat = _sc_gather(idx.reshape(-1), table)
    knn = knn_flat.reshape(b, n * K, 16)

    w1t16 = jnp.pad(W1.T, ((0, 6), (0, 0)))
    return _encode(knn, xyz_t16, feature, w1t16, Wl1.T, Wl2.T,
                   b1[None, :], bl1[None, :], bl2[None, :],
                   Wsc, bsc[:, None])
